# Initial kernel scaffold; baseline (speedup 1.0000x reference)
import jax
import jax.numpy as jnp
from jax.experimental import pallas as pl


def kernel(atomwise_energies, atoms, atom_refs):
    raise NotImplementedError("write your pallas kernel here")


# speedup vs baseline: 23.0500x; 23.0500x over previous
import functools

import jax
import jax.numpy as jnp
from jax import lax
from jax.experimental import pallas as pl
from jax.experimental.pallas import tpu as pltpu
from jax.experimental.pallas import tpu_sc as plsc

N = 100000
LANES = 16
NUM_CORES = 2
NUM_SUBCORES = 16
NUM_WORKERS = NUM_CORES * NUM_SUBCORES
CHUNK = 3136
STEPS = CHUNK // LANES
LAST_BASE = N - CHUNK
TABLE = 100
TABLE_PAD = 128


def _sc_add_refs(energies, atoms, table):
    mesh = plsc.VectorSubcoreMesh(core_axis_name="c", subcore_axis_name="s")

    @functools.partial(
        pl.kernel,
        mesh=mesh,
        out_type=jax.ShapeDtypeStruct((N,), jnp.float32),
        compiler_params=pltpu.CompilerParams(needs_layout_passes=False),
        scratch_types=[
            pltpu.VMEM((TABLE_PAD,), jnp.float32),
            pltpu.VMEM((CHUNK,), jnp.int32),
            pltpu.VMEM((CHUNK,), jnp.float32),
            pltpu.VMEM((CHUNK,), jnp.float32),
            pltpu.SemaphoreType.DMA,
            pltpu.SemaphoreType.DMA,
            pltpu.SemaphoreType.DMA,
        ],
    )
    def k(e_hbm, a_hbm, t_hbm, out_hbm, table_v, idx_v, e_v, o_v,
          sem_t, sem_i, sem_e):
        wid = lax.axis_index("s") * NUM_CORES + lax.axis_index("c")
        base = pl.multiple_of(lax.min(wid * CHUNK, LAST_BASE), 8)
        cp_t = pltpu.async_copy(t_hbm, table_v.at[pl.ds(0, TABLE)], sem_t)
        cp_i = pltpu.async_copy(a_hbm.at[pl.ds(base, CHUNK)], idx_v, sem_i)
        cp_e = pltpu.async_copy(e_hbm.at[pl.ds(base, CHUNK)], e_v, sem_e)
        cp_t.wait()
        cp_i.wait()
        cp_e.wait()

        def body(i, carry):
            off = i * LANES
            iv = idx_v[pl.ds(off, LANES)]
            refs = plsc.load_gather(table_v, [iv])
            o_v[pl.ds(off, LANES)] = e_v[pl.ds(off, LANES)] + refs
            return carry

        lax.fori_loop(0, STEPS, body, 0)
        pltpu.sync_copy(o_v, out_hbm.at[pl.ds(base, CHUNK)])

    return k(energies, atoms, table)


def kernel(atomwise_energies, atoms, atom_refs):
    energies = atomwise_energies.reshape(N)
    table = atom_refs.reshape(TABLE)
    out = _sc_add_refs(energies, atoms.astype(jnp.int32), table)
    return out.reshape(N, 1)


# speedup vs baseline: 23.7516x; 1.0304x over previous
import functools

import jax
import jax.numpy as jnp
from jax import lax
from jax.experimental import pallas as pl
from jax.experimental.pallas import tpu as pltpu
from jax.experimental.pallas import tpu_sc as plsc

N = 100000
LANES = 16
NUM_CORES = 2
NUM_SUBCORES = 16
NUM_WORKERS = NUM_CORES * NUM_SUBCORES
CHUNK = 3136
STEPS = CHUNK // LANES
LAST_BASE = N - CHUNK
TABLE = 100
TABLE_PAD = 128


def _sc_add_refs(energies, atoms, table):
    mesh = plsc.VectorSubcoreMesh(core_axis_name="c", subcore_axis_name="s")

    @functools.partial(
        pl.kernel,
        mesh=mesh,
        out_type=jax.ShapeDtypeStruct((N,), jnp.float32),
        compiler_params=pltpu.CompilerParams(needs_layout_passes=False),
        scratch_types=[
            pltpu.VMEM((TABLE_PAD,), jnp.float32),
            pltpu.VMEM((CHUNK,), jnp.int32),
            pltpu.VMEM((CHUNK,), jnp.float32),
            pltpu.VMEM((CHUNK,), jnp.float32),
            pltpu.SemaphoreType.DMA,
            pltpu.SemaphoreType.DMA,
            pltpu.SemaphoreType.DMA,
            pltpu.SemaphoreType.DMA,
        ],
    )
    def k(e_hbm, a_hbm, t_hbm, out_hbm, table_v, idx_v, e_v, o_v,
          sem_t, sem_i, sem_e, sem_o):
        wid = lax.axis_index("s") * NUM_CORES + lax.axis_index("c")
        base = pl.multiple_of(lax.min(wid * CHUNK, LAST_BASE), 8)
        cp_t = pltpu.async_copy(t_hbm, table_v.at[pl.ds(0, TABLE)], sem_t)
        cp_i = pltpu.async_copy(a_hbm.at[pl.ds(base, CHUNK)], idx_v, sem_i)
        cp_e = pltpu.async_copy(e_hbm.at[pl.ds(base, CHUNK)], e_v, sem_e)
        cp_t.wait()
        cp_i.wait()
        cp_e.wait()

        half = CHUNK // 2

        @plsc.parallel_loop(0, half, LANES, unroll=8)
        def _first(off):
            iv = idx_v[pl.ds(off, LANES)]
            refs = plsc.load_gather(table_v, [iv])
            o_v[pl.ds(off, LANES)] = e_v[pl.ds(off, LANES)] + refs

        cp_o = pltpu.async_copy(
            o_v.at[pl.ds(0, half)], out_hbm.at[pl.ds(base, half)], sem_o)

        @plsc.parallel_loop(half, CHUNK, LANES, unroll=8)
        def _second(off):
            iv = idx_v[pl.ds(off, LANES)]
            refs = plsc.load_gather(table_v, [iv])
            o_v[pl.ds(off, LANES)] = e_v[pl.ds(off, LANES)] + refs

        cp_o.wait()
        pltpu.sync_copy(o_v.at[pl.ds(half, half)],
                        out_hbm.at[pl.ds(base + half, half)])

    return k(energies, atoms, table)


def kernel(atomwise_energies, atoms, atom_refs):
    energies = atomwise_energies.reshape(N)
    table = atom_refs.reshape(TABLE)
    out = _sc_add_refs(energies, atoms.astype(jnp.int32), table)
    return out.reshape(N, 1)
